# baseline (speedup 1.0000x reference)
import functools

import jax
import jax.numpy as jnp
from jax import lax
from jax.experimental import pallas as pl
from jax.experimental.pallas import tpu as pltpu
from jax.experimental.pallas import tpu_sc as plsc

VOCAB = 1000000
EMB_DIM = 64
OUT_DIM = 1000
BATCH = 4096
SEQ = 200

C0 = 96
C1 = 104

_NC = 2
_NS = 16
_NW = _NC * _NS
_ROWS_PER_W = BATCH // _NW


def _sc_body(table_hbm, idx_hbm, out_hbm, idx_v, gA, gB, obuf, sem):
  wid = lax.axis_index("s") * _NC + lax.axis_index("c")
  base = wid * _ROWS_PER_W

  pltpu.sync_copy(idx_hbm.at[pl.ds(base * SEQ, _ROWS_PER_W * SEQ)], idx_v)

  def gather_pair(b, gbuf):
    c0 = pltpu.make_async_copy(
        table_hbm.at[idx_v.at[pl.ds(b * SEQ, C0)]],
        gbuf.at[pl.ds(0, C0)], sem)
    c1 = pltpu.make_async_copy(
        table_hbm.at[idx_v.at[pl.ds(b * SEQ + C0, C1)]],
        gbuf.at[pl.ds(C0, C1)], sem)
    return c0, c1

  def issue(b, gbuf):
    c0, c1 = gather_pair(b, gbuf)
    c0.start()
    c1.start()

  def wait(b, gbuf):
    c0, c1 = gather_pair(b, gbuf)
    c0.wait()
    c1.wait()

  def sum_row(gbuf, b):
    def body(j, accs):
      a0, a1, a2, a3 = accs
      r0 = j * 8
      for jj in range(8):
        r = r0 + jj
        a0 = a0 + gbuf[r, pl.ds(0, 16)]
        a1 = a1 + gbuf[r, pl.ds(16, 16)]
        a2 = a2 + gbuf[r, pl.ds(32, 16)]
        a3 = a3 + gbuf[r, pl.ds(48, 16)]
      return (a0, a1, a2, a3)

    z = jnp.zeros((16,), jnp.float32)
    a0, a1, a2, a3 = lax.fori_loop(0, SEQ // 8, body, (z, z, z, z))
    obuf[b, pl.ds(0, 16)] = a0
    obuf[b, pl.ds(16, 16)] = a1
    obuf[b, pl.ds(32, 16)] = a2
    obuf[b, pl.ds(48, 16)] = a3

  issue(0, gA)

  def outer(i, carry):
    b0 = 2 * i
    b1 = b0 + 1
    issue(b1, gB)
    wait(b0, gA)
    sum_row(gA, b0)

    @pl.when(i < _ROWS_PER_W // 2 - 1)
    def _():
      issue(b0 + 2, gA)

    wait(b1, gB)
    sum_row(gB, b1)
    return carry

  lax.fori_loop(0, _ROWS_PER_W // 2, outer, 0)

  pltpu.sync_copy(obuf, out_hbm.at[pl.ds(base, _ROWS_PER_W)])


def _sc_gather_sum(table, idx):
  mesh = plsc.VectorSubcoreMesh(core_axis_name="c", subcore_axis_name="s")
  f = functools.partial(
      pl.kernel,
      mesh=mesh,
      compiler_params=pltpu.CompilerParams(use_tc_tiling_on_sc=False),
      out_type=jax.ShapeDtypeStruct((BATCH, EMB_DIM), jnp.float32),
      scratch_types=[
          pltpu.VMEM((_ROWS_PER_W * SEQ,), jnp.int32),
          pltpu.VMEM((SEQ, EMB_DIM), jnp.float32),
          pltpu.VMEM((SEQ, EMB_DIM), jnp.float32),
          pltpu.VMEM((_ROWS_PER_W, EMB_DIM), jnp.float32),
          pltpu.SemaphoreType.DMA,
      ],
  )(_sc_body)
  return f(table, idx)


def _fc_body(emb_ref, w_ref, b_ref, out_ref):
  e = emb_ref[...]
  ss = jnp.sum(e * e, axis=1, keepdims=True)
  scale = 1.0 / jnp.maximum(jnp.sqrt(ss), 1e-12)
  en = e * scale
  acc = lax.dot_general(
      en, w_ref[...], (((1,), (1,)), ((), ())),
      preferred_element_type=jnp.float32,
      precision=lax.Precision.HIGHEST)
  out_ref[...] = acc + b_ref[...]


def _norm_fc(emb, W, b):
  BB = 512
  return pl.pallas_call(
      _fc_body,
      grid=(BATCH // BB,),
      in_specs=[
          pl.BlockSpec((BB, EMB_DIM), lambda i: (i, 0)),
          pl.BlockSpec((OUT_DIM, EMB_DIM), lambda i: (0, 0)),
          pl.BlockSpec((1, OUT_DIM), lambda i: (0, 0)),
      ],
      out_specs=pl.BlockSpec((BB, OUT_DIM), lambda i: (i, 0)),
      out_shape=jax.ShapeDtypeStruct((BATCH, OUT_DIM), jnp.float32),
  )(emb, W, b.reshape(1, OUT_DIM))


def kernel(name_idxs, name_len, desc_idxs, desc_len, union_idxs, union_len,
           table, W, b):
  idx = union_idxs.astype(jnp.int32).reshape(BATCH * SEQ)
  emb_sum = _sc_gather_sum(table, idx)
  return _norm_fc(emb_sum, W, b)
